# baseline (speedup 1.0000x reference)
import jax
import jax.numpy as jnp
from jax import lax
from jax.experimental import pallas as pl
from jax.experimental.pallas import tpu as pltpu
from jax.experimental.pallas import tpu_sc as plsc

N = 10000
E = 320000
FH = 80
F2 = 48

NC = 2
NS = 16
NW = NC * NS
N_PAD = 10112
STRIPE = N_PAD // NS

C1 = 56
PH1 = 179
CPW1 = 2 * PH1
E1 = NS * CPW1 * C1

C2 = 108
CPW2 = 93
E2 = NW * CPW2 * C2


def _pipeline(n, feat_sh, acc_sh, src_v, dst_v, rows_a, rows_b,
              sem_ga, sem_gb, sem_sa, sem_sb):
    def gather_start(c, buf, sem):
        pltpu.async_copy(feat_sh.at[src_v.at[c]], buf, sem)

    def gather_wait(c, buf, sem):
        pltpu.make_async_copy(feat_sh.at[src_v.at[c]], buf, sem).wait()

    def scatter_start(c, buf, sem):
        pltpu.async_copy(buf, acc_sh.at[dst_v.at[c]], sem, add=True)

    def scatter_wait(c, buf, sem):
        pltpu.make_async_copy(buf, acc_sh.at[dst_v.at[c]], sem).wait()

    gather_start(0, rows_a, sem_ga)
    gather_start(1, rows_b, sem_gb)

    @pl.loop(0, n // 2)
    def _(p):
        c = 2 * p
        gather_wait(c, rows_a, sem_ga)
        scatter_start(c, rows_a, sem_sa)
        gather_wait(c + 1, rows_b, sem_gb)
        scatter_start(c + 1, rows_b, sem_sb)
        scatter_wait(c, rows_a, sem_sa)
        gather_start(c + 2, rows_a, sem_ga)
        scatter_wait(c + 1, rows_b, sem_sb)

        @pl.when(c + 3 < n)
        def _():
            gather_start(c + 3, rows_b, sem_gb)

    gather_wait(n - 1, rows_a, sem_ga)
    pltpu.sync_copy(rows_a, acc_sh.at[dst_v.at[n - 1]], add=True)


def _pipeline3(n, feat_sh, acc_sh, src_v, dst_v, bufs, gsems, ssems):
    def gather_start(c, b):
        pltpu.async_copy(feat_sh.at[src_v.at[c]], bufs[b], gsems[b])

    def gather_wait(c, b):
        pltpu.make_async_copy(feat_sh.at[src_v.at[c]], bufs[b],
                              gsems[b]).wait()

    def scatter_start(c, b):
        pltpu.async_copy(bufs[b], acc_sh.at[dst_v.at[c]], ssems[b], add=True)

    def scatter_wait(c, b):
        pltpu.make_async_copy(bufs[b], acc_sh.at[dst_v.at[c]],
                              ssems[b]).wait()

    for b in range(3):
        gather_start(b, b)

    @pl.loop(0, n // 3)
    def _(p):
        c = 3 * p
        for b in range(3):
            gather_wait(c + b, b)
            scatter_start(c + b, b)
        for b in range(3):
            scatter_wait(c + b, b)

            @pl.when(c + 3 + b < n)
            def _():
                gather_start(c + 3 + b, b)


def _mesh():
    return plsc.VectorSubcoreMesh(core_axis_name="c", subcore_axis_name="s")


def _layer1_seg_sum():
    out_type = jax.ShapeDtypeStruct((NC, N_PAD, FH), jnp.float32)
    scratch = [
        pltpu.VMEM((PH1, C1), jnp.int32),
        pltpu.VMEM((PH1, C1), jnp.int32),
        pltpu.VMEM((C1, FH), jnp.float32),
        pltpu.VMEM((C1, FH), jnp.float32),
        pltpu.VMEM_SHARED((N_PAD, FH), jnp.float32),
        pltpu.VMEM_SHARED((N_PAD, FH), jnp.float32),
        pltpu.SemaphoreType.DMA,
        pltpu.SemaphoreType.DMA,
        pltpu.SemaphoreType.DMA,
        pltpu.SemaphoreType.DMA,
    ]

    def body(xh_hbm, src_hbm, dst_hbm, zeros_hbm, sum_hbm,
             src_v, dst_v, rows_a, rows_b, feat_sh, acc_sh,
             sem_ga, sem_gb, sem_sa, sem_sb):
        cid = lax.axis_index("c")
        sid = lax.axis_index("s")

        base_r = sid * STRIPE
        pltpu.sync_copy(zeros_hbm, acc_sh.at[pl.ds(base_r, STRIPE)])
        pltpu.sync_copy(xh_hbm.at[cid].at[pl.ds(base_r, STRIPE)],
                        feat_sh.at[pl.ds(base_r, STRIPE)])

        for ph in range(2):
            pltpu.sync_copy(src_hbm.at[sid, ph], src_v)
            pltpu.sync_copy(dst_hbm.at[sid, ph], dst_v)
            if ph == 0:
                plsc.subcore_barrier()
            _pipeline(PH1, feat_sh, acc_sh, src_v, dst_v, rows_a, rows_b,
                      sem_ga, sem_gb, sem_sa, sem_sb)

        plsc.subcore_barrier()
        pltpu.sync_copy(acc_sh.at[pl.ds(base_r, STRIPE)],
                        sum_hbm.at[cid].at[pl.ds(base_r, STRIPE)])

    return pl.kernel(
        body, out_type=out_type, mesh=_mesh(), scratch_types=scratch,
        compiler_params=pltpu.CompilerParams(use_tc_tiling_on_sc=False))


def _layer2_seg_sum():
    out_type = jax.ShapeDtypeStruct((NC, N_PAD, F2), jnp.float32)
    scratch = [
        pltpu.VMEM((CPW2, C2), jnp.int32),
        pltpu.VMEM((CPW2, C2), jnp.int32),
        pltpu.VMEM((C2, F2), jnp.float32),
        pltpu.VMEM((C2, F2), jnp.float32),
        pltpu.VMEM((C2, F2), jnp.float32),
        pltpu.VMEM_SHARED((N_PAD, F2), jnp.float32),
        pltpu.VMEM_SHARED((N_PAD, F2), jnp.float32),
        pltpu.SemaphoreType.DMA,
        pltpu.SemaphoreType.DMA,
        pltpu.SemaphoreType.DMA,
        pltpu.SemaphoreType.DMA,
        pltpu.SemaphoreType.DMA,
        pltpu.SemaphoreType.DMA,
    ]

    def body(feat_hbm, src_hbm, dst_hbm, zeros_hbm, sum_hbm,
             src_v, dst_v, rows_a, rows_b, rows_c, feat_sh, acc_sh,
             sem_ga, sem_gb, sem_gc, sem_sa, sem_sb, sem_sc):
        cid = lax.axis_index("c")
        sid = lax.axis_index("s")
        wid = cid * NS + sid

        base_r = sid * STRIPE
        pltpu.sync_copy(zeros_hbm, acc_sh.at[pl.ds(base_r, STRIPE)])
        pltpu.sync_copy(feat_hbm.at[pl.ds(base_r, STRIPE)],
                        feat_sh.at[pl.ds(base_r, STRIPE)])
        pltpu.sync_copy(src_hbm.at[wid], src_v)
        pltpu.sync_copy(dst_hbm.at[wid], dst_v)
        plsc.subcore_barrier()

        _pipeline3(CPW2, feat_sh, acc_sh, src_v, dst_v,
                   (rows_a, rows_b, rows_c),
                   (sem_ga, sem_gb, sem_gc), (sem_sa, sem_sb, sem_sc))

        plsc.subcore_barrier()
        pltpu.sync_copy(acc_sh.at[pl.ds(base_r, STRIPE)],
                        sum_hbm.at[cid].at[pl.ds(base_r, STRIPE)])

    return pl.kernel(
        body, out_type=out_type, mesh=_mesh(), scratch_types=scratch,
        compiler_params=pltpu.CompilerParams(use_tc_tiling_on_sc=False))


_seg_sum_l1 = _layer1_seg_sum()
_seg_sum_l2 = _layer2_seg_sum()

_BM = 1000


def _layer1_body(p_ref, w1_ref, b1_ref, w2_ref, z_ref, r_ref):
    feats = jnp.concatenate([p_ref[0, :, :64], p_ref[1, :, :64]], axis=1)
    deg = p_ref[0, :, 64:65]
    recip = 1.0 / jnp.maximum(deg, 1.0)
    mean = feats * recip
    h = jnp.dot(mean, w1_ref[...], preferred_element_type=jnp.float32)
    h = jnp.maximum(h + b1_ref[...][None, :], 0.0)
    z_ref[...] = jnp.dot(h, w2_ref[...], preferred_element_type=jnp.float32)
    r_ref[...] = jnp.broadcast_to(recip, (r_ref.shape[0], 8))


def _layer2_body(p_ref, r_ref, b2_ref, o_ref):
    msum = p_ref[0] + p_ref[1]
    mean = msum * r_ref[:, 0:1]
    o_ref[...] = mean[:, :40] + b2_ref[...][None, :]


def kernel(x, edge_index, W1, b1, W2, b2):
    src = edge_index[0].astype(jnp.int32)
    dst = edge_index[1].astype(jnp.int32)
    x = x.astype(jnp.float32)

    xp = jnp.zeros((N_PAD, 128), jnp.float32).at[:N].set(x)
    ones = jnp.ones((N_PAD, 16), jnp.float32)
    xh = jnp.stack([
        jnp.concatenate([xp[:, :64], ones], axis=1),
        jnp.concatenate([xp[:, 64:128], ones], axis=1),
    ])

    pad1 = E1 - E
    pad2 = E2 - E
    scr = N + jnp.arange(max(pad1, pad2), dtype=jnp.int32) % (N_PAD - N)
    src4 = jnp.concatenate(
        [src, jnp.zeros((pad1,), jnp.int32)]).reshape(NS, 2, PH1, C1)
    dst4 = jnp.concatenate([dst, scr[:pad1]]).reshape(NS, 2, PH1, C1)

    msum = _seg_sum_l1(xh, src4, dst4, jnp.zeros((STRIPE, FH), jnp.float32))

    w2p = jnp.zeros((128, F2), jnp.float32).at[:, :40].set(W2)
    bmb = N_PAD // 16
    z, recip = pl.pallas_call(
        _layer1_body,
        grid=(16,),
        in_specs=[
            pl.BlockSpec((NC, bmb, FH), lambda i: (0, i, 0)),
            pl.BlockSpec((128, 128), lambda i: (0, 0)),
            pl.BlockSpec((128,), lambda i: (0,)),
            pl.BlockSpec((128, F2), lambda i: (0, 0)),
        ],
        out_specs=[
            pl.BlockSpec((bmb, F2), lambda i: (i, 0)),
            pl.BlockSpec((bmb, 8), lambda i: (i, 0)),
        ],
        out_shape=[
            jax.ShapeDtypeStruct((N_PAD, F2), jnp.float32),
            jax.ShapeDtypeStruct((N_PAD, 8), jnp.float32),
        ],
    )(msum, W1, b1, w2p)

    src3 = jnp.concatenate(
        [src, jnp.zeros((pad2,), jnp.int32)]).reshape(NW, CPW2, C2)
    dst3 = jnp.concatenate([dst, scr[:pad2]]).reshape(NW, CPW2, C2)

    msum2 = _seg_sum_l2(z, src3, dst3, jnp.zeros((STRIPE, F2), jnp.float32))

    out = pl.pallas_call(
        _layer2_body,
        grid=(N // _BM,),
        in_specs=[
            pl.BlockSpec((NC, _BM, F2), lambda i: (0, i, 0)),
            pl.BlockSpec((_BM, 8), lambda i: (i, 0)),
            pl.BlockSpec((40,), lambda i: (0,)),
        ],
        out_specs=pl.BlockSpec((_BM, 40), lambda i: (i, 0)),
        out_shape=jax.ShapeDtypeStruct((N, 40), jnp.float32),
    )(msum2, recip, b2)
    return out


# speedup vs baseline: 1.0236x; 1.0236x over previous
import jax
import jax.numpy as jnp
from jax import lax
from jax.experimental import pallas as pl
from jax.experimental.pallas import tpu as pltpu
from jax.experimental.pallas import tpu_sc as plsc

N = 10000
E = 320000
FH = 80
F2 = 48

NC = 2
NS = 16
NW = NC * NS
N_PAD = 10112
STRIPE = N_PAD // NS

C1 = 56
PH1 = 179
CPW1 = 2 * PH1
E1 = NS * CPW1 * C1

C2 = 112
CPW2 = 91
E2 = NW * CPW2 * C2


def _pipeline(n, feat_sh, acc_sh, src_v, dst_v, rows_a, rows_b,
              sem_ga, sem_gb, sem_sa, sem_sb):
    def gather_start(c, buf, sem):
        pltpu.async_copy(feat_sh.at[src_v.at[c]], buf, sem)

    def gather_wait(c, buf, sem):
        pltpu.make_async_copy(feat_sh.at[src_v.at[c]], buf, sem).wait()

    def scatter_start(c, buf, sem):
        pltpu.async_copy(buf, acc_sh.at[dst_v.at[c]], sem, add=True)

    def scatter_wait(c, buf, sem):
        pltpu.make_async_copy(buf, acc_sh.at[dst_v.at[c]], sem).wait()

    gather_start(0, rows_a, sem_ga)
    gather_start(1, rows_b, sem_gb)

    @pl.loop(0, n // 2)
    def _(p):
        c = 2 * p
        gather_wait(c, rows_a, sem_ga)
        scatter_start(c, rows_a, sem_sa)
        gather_wait(c + 1, rows_b, sem_gb)
        scatter_start(c + 1, rows_b, sem_sb)
        scatter_wait(c, rows_a, sem_sa)
        gather_start(c + 2, rows_a, sem_ga)
        scatter_wait(c + 1, rows_b, sem_sb)

        @pl.when(c + 3 < n)
        def _():
            gather_start(c + 3, rows_b, sem_gb)

    gather_wait(n - 1, rows_a, sem_ga)
    pltpu.sync_copy(rows_a, acc_sh.at[dst_v.at[n - 1]], add=True)


def _mesh():
    return plsc.VectorSubcoreMesh(core_axis_name="c", subcore_axis_name="s")


def _layer1_seg_sum():
    out_type = jax.ShapeDtypeStruct((NC, N_PAD, FH), jnp.float32)
    scratch = [
        pltpu.VMEM((PH1, C1), jnp.int32),
        pltpu.VMEM((PH1, C1), jnp.int32),
        pltpu.VMEM((C1, FH), jnp.float32),
        pltpu.VMEM((C1, FH), jnp.float32),
        pltpu.VMEM_SHARED((N_PAD, FH), jnp.float32),
        pltpu.VMEM_SHARED((N_PAD, FH), jnp.float32),
        pltpu.SemaphoreType.DMA,
        pltpu.SemaphoreType.DMA,
        pltpu.SemaphoreType.DMA,
        pltpu.SemaphoreType.DMA,
    ]

    def body(xh_hbm, src_hbm, dst_hbm, zeros_hbm, sum_hbm,
             src_v, dst_v, rows_a, rows_b, feat_sh, acc_sh,
             sem_ga, sem_gb, sem_sa, sem_sb):
        cid = lax.axis_index("c")
        sid = lax.axis_index("s")

        base_r = sid * STRIPE
        pltpu.sync_copy(zeros_hbm, acc_sh.at[pl.ds(base_r, STRIPE)])
        pltpu.sync_copy(xh_hbm.at[cid].at[pl.ds(base_r, STRIPE)],
                        feat_sh.at[pl.ds(base_r, STRIPE)])

        for ph in range(2):
            pltpu.sync_copy(src_hbm.at[sid, ph], src_v)
            pltpu.sync_copy(dst_hbm.at[sid, ph], dst_v)
            if ph == 0:
                plsc.subcore_barrier()
            _pipeline(PH1, feat_sh, acc_sh, src_v, dst_v, rows_a, rows_b,
                      sem_ga, sem_gb, sem_sa, sem_sb)

        plsc.subcore_barrier()
        pltpu.sync_copy(acc_sh.at[pl.ds(base_r, STRIPE)],
                        sum_hbm.at[cid].at[pl.ds(base_r, STRIPE)])

    return pl.kernel(
        body, out_type=out_type, mesh=_mesh(), scratch_types=scratch,
        compiler_params=pltpu.CompilerParams(use_tc_tiling_on_sc=False))


def _layer2_seg_sum():
    out_type = jax.ShapeDtypeStruct((NC, N_PAD, F2), jnp.float32)
    scratch = [
        pltpu.VMEM((CPW2, C2), jnp.int32),
        pltpu.VMEM((CPW2, C2), jnp.int32),
        pltpu.VMEM((C2, F2), jnp.float32),
        pltpu.VMEM((C2, F2), jnp.float32),
        pltpu.VMEM_SHARED((N_PAD, F2), jnp.float32),
        pltpu.VMEM_SHARED((N_PAD, F2), jnp.float32),
        pltpu.SemaphoreType.DMA,
        pltpu.SemaphoreType.DMA,
        pltpu.SemaphoreType.DMA,
        pltpu.SemaphoreType.DMA,
    ]

    def body(feat_hbm, src_hbm, dst_hbm, zeros_hbm, sum_hbm,
             src_v, dst_v, rows_a, rows_b, feat_sh, acc_sh,
             sem_ga, sem_gb, sem_sa, sem_sb):
        cid = lax.axis_index("c")
        sid = lax.axis_index("s")
        wid = cid * NS + sid

        base_r = sid * STRIPE
        pltpu.sync_copy(zeros_hbm, acc_sh.at[pl.ds(base_r, STRIPE)])
        pltpu.sync_copy(feat_hbm.at[pl.ds(base_r, STRIPE)],
                        feat_sh.at[pl.ds(base_r, STRIPE)])
        pltpu.sync_copy(src_hbm.at[wid], src_v)
        pltpu.sync_copy(dst_hbm.at[wid], dst_v)
        plsc.subcore_barrier()

        _pipeline(CPW2, feat_sh, acc_sh, src_v, dst_v, rows_a, rows_b,
                  sem_ga, sem_gb, sem_sa, sem_sb)

        plsc.subcore_barrier()
        pltpu.sync_copy(acc_sh.at[pl.ds(base_r, STRIPE)],
                        sum_hbm.at[cid].at[pl.ds(base_r, STRIPE)])

    return pl.kernel(
        body, out_type=out_type, mesh=_mesh(), scratch_types=scratch,
        compiler_params=pltpu.CompilerParams(use_tc_tiling_on_sc=False))


_seg_sum_l1 = _layer1_seg_sum()
_seg_sum_l2 = _layer2_seg_sum()

_BM = 1000


def _layer1_body(p_ref, w1_ref, b1_ref, w2_ref, z_ref, r_ref):
    feats = jnp.concatenate([p_ref[0, :, :64], p_ref[1, :, :64]], axis=1)
    deg = p_ref[0, :, 64:65]
    recip = 1.0 / jnp.maximum(deg, 1.0)
    mean = feats * recip
    h = jnp.dot(mean, w1_ref[...], preferred_element_type=jnp.float32)
    h = jnp.maximum(h + b1_ref[...][None, :], 0.0)
    z_ref[...] = jnp.dot(h, w2_ref[...], preferred_element_type=jnp.float32)
    r_ref[...] = jnp.broadcast_to(recip, (r_ref.shape[0], 8))


def _layer2_body(p_ref, r_ref, b2_ref, o_ref):
    msum = p_ref[0] + p_ref[1]
    mean = msum * r_ref[:, 0:1]
    o_ref[...] = mean[:, :40] + b2_ref[...][None, :]


def kernel(x, edge_index, W1, b1, W2, b2):
    src = edge_index[0].astype(jnp.int32)
    dst = edge_index[1].astype(jnp.int32)
    x = x.astype(jnp.float32)

    xp = jnp.zeros((N_PAD, 128), jnp.float32).at[:N].set(x)
    ones = jnp.ones((N_PAD, 16), jnp.float32)
    xh = jnp.stack([
        jnp.concatenate([xp[:, :64], ones], axis=1),
        jnp.concatenate([xp[:, 64:128], ones], axis=1),
    ])

    pad1 = E1 - E
    pad2 = E2 - E
    scr = N + jnp.arange(max(pad1, pad2), dtype=jnp.int32) % (N_PAD - N)
    src4 = jnp.concatenate(
        [src, jnp.zeros((pad1,), jnp.int32)]).reshape(NS, 2, PH1, C1)
    dst4 = jnp.concatenate([dst, scr[:pad1]]).reshape(NS, 2, PH1, C1)

    msum = _seg_sum_l1(xh, src4, dst4, jnp.zeros((STRIPE, FH), jnp.float32))

    w2p = jnp.zeros((128, F2), jnp.float32).at[:, :40].set(W2)
    bmb = N_PAD // 16
    z, recip = pl.pallas_call(
        _layer1_body,
        grid=(16,),
        in_specs=[
            pl.BlockSpec((NC, bmb, FH), lambda i: (0, i, 0)),
            pl.BlockSpec((128, 128), lambda i: (0, 0)),
            pl.BlockSpec((128,), lambda i: (0,)),
            pl.BlockSpec((128, F2), lambda i: (0, 0)),
        ],
        out_specs=[
            pl.BlockSpec((bmb, F2), lambda i: (i, 0)),
            pl.BlockSpec((bmb, 8), lambda i: (i, 0)),
        ],
        out_shape=[
            jax.ShapeDtypeStruct((N_PAD, F2), jnp.float32),
            jax.ShapeDtypeStruct((N_PAD, 8), jnp.float32),
        ],
    )(msum, W1, b1, w2p)

    src3 = jnp.concatenate(
        [src, jnp.zeros((pad2,), jnp.int32)]).reshape(NW, CPW2, C2)
    dst3 = jnp.concatenate([dst, scr[:pad2]]).reshape(NW, CPW2, C2)

    msum2 = _seg_sum_l2(z, src3, dst3, jnp.zeros((STRIPE, F2), jnp.float32))

    out = pl.pallas_call(
        _layer2_body,
        grid=(N // _BM,),
        in_specs=[
            pl.BlockSpec((NC, _BM, F2), lambda i: (0, i, 0)),
            pl.BlockSpec((_BM, 8), lambda i: (i, 0)),
            pl.BlockSpec((40,), lambda i: (0,)),
        ],
        out_specs=pl.BlockSpec((_BM, 40), lambda i: (i, 0)),
        out_shape=jax.ShapeDtypeStruct((N, 40), jnp.float32),
    )(msum2, recip, b2)
    return out
